# Initial kernel scaffold; baseline (speedup 1.0000x reference)
import jax
import jax.numpy as jnp
from jax.experimental import pallas as pl


def kernel(x, H0, H1, H2, H3, H4, b0, b1, b2, b3, b4, a0, a1, a2, a3):
    raise NotImplementedError("write your pallas kernel here")


# speedup vs baseline: 1.8248x; 1.8248x over previous
import functools

import jax
import jax.numpy as jnp
from jax.experimental import pallas as pl

_CB = 8


def _dense_body(x_ref, h0, h1, h2, h3, h4, c0, c1, c2, c3, c4,
                t0, t1, t2, t3, xh_ref, lk_ref):
    v = x_ref[...]
    vh = jnp.round(v)
    xh_ref[...] = vh

    w0 = jax.nn.softplus(h0[...])
    w1 = jax.nn.softplus(h1[...])
    w2 = jax.nn.softplus(h2[...])
    w3 = jax.nn.softplus(h3[...])
    w4 = jax.nn.softplus(h4[...])
    f0 = jnp.tanh(t0[...])
    f1 = jnp.tanh(t1[...])
    f2 = jnp.tanh(t2[...])
    f3 = jnp.tanh(t3[...])
    bb0, bb1, bb2, bb3 = c0[...], c1[...], c2[...], c3[...]
    bb4 = c4[...]

    def sl(w, j):
        return w[:, :, j:j + 1]

    def logits(u):
        l = [sl(w0, j) * u + sl(bb0, j) for j in range(3)]
        l = [l[j] + sl(f0, j) * jnp.tanh(l[j]) for j in range(3)]
        for (w, bb, f) in ((w1, bb1, f1), (w2, bb2, f2), (w3, bb3, f3)):
            nl = [sl(w, 3 * j) * l[0] + sl(w, 3 * j + 1) * l[1]
                  + sl(w, 3 * j + 2) * l[2] + sl(bb, j) for j in range(3)]
            l = [nl[j] + sl(f, j) * jnp.tanh(nl[j]) for j in range(3)]
        return sl(w4, 0) * l[0] + sl(w4, 1) * l[1] + sl(w4, 2) * l[2] + sl(bb4, 0)

    lower = logits(vh - 0.5)
    upper = logits(vh + 0.5)
    s = -jnp.sign(lower + upper)
    lk = jnp.abs(jax.nn.sigmoid(s * upper) - jax.nn.sigmoid(s * lower))
    lk_ref[...] = jnp.maximum(lk, 1e-9)


@jax.jit
def kernel(x, H0, H1, H2, H3, H4, b0, b1, b2, b3, b4, a0, a1, a2, a3):
    B, C, H, W = x.shape
    N = H * W
    xr = x.reshape(B, C, N)
    ws = [H0.reshape(1, C, 3), H1.reshape(1, C, 9), H2.reshape(1, C, 9),
          H3.reshape(1, C, 9), H4.reshape(1, C, 3),
          b0.reshape(1, C, 3), b1.reshape(1, C, 3), b2.reshape(1, C, 3),
          b3.reshape(1, C, 3), b4.reshape(1, C, 1),
          a0.reshape(1, C, 3), a1.reshape(1, C, 3), a2.reshape(1, C, 3),
          a3.reshape(1, C, 3)]
    w_specs = [pl.BlockSpec((1, _CB, w.shape[2]), lambda c: (0, c, 0))
               for w in ws]
    xh, lk = pl.pallas_call(
        _dense_body,
        grid=(C // _CB,),
        in_specs=[pl.BlockSpec((B, _CB, N), lambda c: (0, c, 0))] + w_specs,
        out_specs=[pl.BlockSpec((B, _CB, N), lambda c: (0, c, 0))] * 2,
        out_shape=[jax.ShapeDtypeStruct((B, C, N), jnp.float32)] * 2,
    )(xr, *ws)
    return xh.reshape(B, C, H, W), lk.reshape(B, C, H, W)
